# Initial kernel scaffold; baseline (speedup 1.0000x reference)
import jax
import jax.numpy as jnp
from jax.experimental import pallas as pl


def kernel(array, length_of_link, node_at_link_head, node_at_link_tail, links_at_node):
    raise NotImplementedError("write your pallas kernel here")


# speedup vs baseline: 68.8330x; 68.8330x over previous
import functools

import jax
import jax.numpy as jnp
from jax import lax
from jax.experimental import pallas as pl
from jax.experimental.pallas import tpu as pltpu
from jax.experimental.pallas import tpu_sc as plsc

N = 100000
L = 200000
K = 4

NC = 2
NS = 16
NW = NC * NS

CL = 6256
LP = NW * CL
CN = 3136
NP = NW * CN

_mesh = plsc.VectorSubcoreMesh(core_axis_name="c", subcore_axis_name="s")


def _wid():
    return lax.axis_index("s") * NC + lax.axis_index("c")


@functools.partial(
    pl.kernel,
    out_type=jax.ShapeDtypeStruct((LP,), jnp.float32),
    mesh=_mesh,
    scratch_types=[
        pltpu.VMEM((CL,), jnp.int32),
        pltpu.VMEM((CL,), jnp.int32),
        pltpu.VMEM((CL,), jnp.float32),
        pltpu.VMEM((CL,), jnp.float32),
        pltpu.VMEM((CL,), jnp.float32),
        pltpu.VMEM((CL,), jnp.float32),
        pltpu.SemaphoreType.DMA,
        pltpu.SemaphoreType.DMA,
        pltpu.SemaphoreType.DMA,
    ],
)
def _grad_kernel(head_hbm, tail_hbm, len_hbm, array_hbm, grad_hbm,
                 head_v, tail_v, len_v, hval_v, tval_v, grad_v,
                 sem0, sem1, sem2):
    base = _wid() * CL
    cp_h = pltpu.async_copy(head_hbm.at[pl.ds(base, CL)], head_v, sem0)
    cp_t = pltpu.async_copy(tail_hbm.at[pl.ds(base, CL)], tail_v, sem1)
    cp_l = pltpu.async_copy(len_hbm.at[pl.ds(base, CL)], len_v, sem2)
    cp_h.wait()
    g_h = pltpu.async_copy(array_hbm.at[head_v], hval_v, sem0)
    cp_t.wait()
    g_t = pltpu.async_copy(array_hbm.at[tail_v], tval_v, sem1)
    cp_l.wait()
    g_h.wait()
    g_t.wait()

    def body(i, carry):
        ds = pl.ds(i * 16, 16)
        grad_v[ds] = (hval_v[ds] - tval_v[ds]) / len_v[ds]
        return carry

    lax.fori_loop(0, CL // 16, body, 0)
    pltpu.sync_copy(grad_v, grad_hbm.at[pl.ds(base, CL)])


@functools.partial(
    pl.kernel,
    out_type=jax.ShapeDtypeStruct((NP,), jnp.float32),
    mesh=_mesh,
    scratch_types=[
        [pltpu.VMEM((CN,), jnp.int32) for _ in range(K)],
        [pltpu.VMEM((CN,), jnp.float32) for _ in range(K)],
        pltpu.VMEM((CN,), jnp.float32),
        [pltpu.SemaphoreType.DMA for _ in range(K)],
    ],
)
def _slope_kernel(grad_hbm, linksT_hbm, out_hbm, links_v, g_v, out_v, sems):
    base = _wid() * CN
    idx_cps = [
        pltpu.async_copy(linksT_hbm.at[pl.ds(j * NP + base, CN)], links_v[j], sems[j])
        for j in range(K)
    ]
    g_cps = []
    for j in range(K):
        idx_cps[j].wait()
        g_cps.append(pltpu.async_copy(grad_hbm.at[links_v[j]], g_v[j], sems[j]))
    for cp in g_cps:
        cp.wait()

    def body(i, carry):
        ds = pl.ds(i * 16, 16)
        out_v[ds] = ((g_v[0][ds] + g_v[1][ds]) + (g_v[2][ds] + g_v[3][ds])) * 0.25
        return carry

    lax.fori_loop(0, CN // 16, body, 0)
    pltpu.sync_copy(out_v, out_hbm.at[pl.ds(base, CN)])


def kernel(array, length_of_link, node_at_link_head, node_at_link_tail, links_at_node):
    head = jnp.pad(node_at_link_head, (0, LP - L))
    tail = jnp.pad(node_at_link_tail, (0, LP - L))
    lens = jnp.pad(length_of_link, (0, LP - L), constant_values=1.0)
    linksT = jnp.pad(links_at_node, ((0, NP - N), (0, 0))).T.reshape(-1)
    grad = _grad_kernel(head, tail, lens, array)
    slope = _slope_kernel(grad, linksT)
    return slope[:N]


# speedup vs baseline: 79.1603x; 1.1500x over previous
import functools

import jax
import jax.numpy as jnp
from jax import lax
from jax.experimental import pallas as pl
from jax.experimental.pallas import tpu as pltpu
from jax.experimental.pallas import tpu_sc as plsc

N = 100000
L = 200000
K = 4

NC = 2
NS = 16
NW = NC * NS

CL = 6256
CN = 3136

_mesh = plsc.VectorSubcoreMesh(core_axis_name="c", subcore_axis_name="s")
_params = pltpu.CompilerParams(needs_layout_passes=False)


def _wid():
    return lax.axis_index("s") * NC + lax.axis_index("c")


@functools.partial(
    pl.kernel,
    out_type=jax.ShapeDtypeStruct((L,), jnp.float32),
    mesh=_mesh,
    compiler_params=_params,
    scratch_types=[
        pltpu.VMEM((N,), jnp.float32),
        pltpu.VMEM((CL,), jnp.int32),
        pltpu.VMEM((CL,), jnp.int32),
        pltpu.VMEM((CL,), jnp.float32),
        pltpu.VMEM((CL,), jnp.float32),
        pltpu.SemaphoreType.DMA,
        pltpu.SemaphoreType.DMA,
        pltpu.SemaphoreType.DMA,
        pltpu.SemaphoreType.DMA,
    ],
)
def _grad_kernel(head_hbm, tail_hbm, len_hbm, array_hbm, grad_hbm,
                 arr_v, head_v, tail_v, len_v, grad_v,
                 sem0, sem1, sem2, sem3):
    base = jnp.minimum(_wid() * CL, L - CL)
    cp_a = pltpu.async_copy(array_hbm, arr_v, sem3)
    cp_h = pltpu.async_copy(head_hbm.at[pl.ds(base, CL)], head_v, sem0)
    cp_t = pltpu.async_copy(tail_hbm.at[pl.ds(base, CL)], tail_v, sem1)
    cp_l = pltpu.async_copy(len_hbm.at[pl.ds(base, CL)], len_v, sem2)
    cp_a.wait()
    cp_h.wait()
    cp_t.wait()
    cp_l.wait()

    def body(i, carry):
        ds = pl.ds(i * 16, 16)
        a_h = plsc.load_gather(arr_v, [head_v[ds]])
        a_t = plsc.load_gather(arr_v, [tail_v[ds]])
        grad_v[ds] = (a_h - a_t) / len_v[ds]
        return carry

    lax.fori_loop(0, CL // 16, body, 0)
    pltpu.sync_copy(grad_v, grad_hbm.at[pl.ds(base, CL)])


@functools.partial(
    pl.kernel,
    out_type=jax.ShapeDtypeStruct((N,), jnp.float32),
    mesh=_mesh,
    compiler_params=_params,
    scratch_types=[
        [pltpu.VMEM((CN,), jnp.int32) for _ in range(K)],
        [pltpu.VMEM((CN,), jnp.float32) for _ in range(K)],
        pltpu.VMEM((CN,), jnp.float32),
        [pltpu.SemaphoreType.DMA for _ in range(K)],
    ],
)
def _slope_kernel(grad_hbm, linksT_hbm, out_hbm, links_v, g_v, out_v, sems):
    base = jnp.minimum(_wid() * CN, N - CN)
    idx_cps = [
        pltpu.async_copy(linksT_hbm.at[pl.ds(j * N + base, CN)], links_v[j], sems[j])
        for j in range(K)
    ]
    g_cps = []
    for j in range(K):
        idx_cps[j].wait()
        g_cps.append(pltpu.async_copy(grad_hbm.at[links_v[j]], g_v[j], sems[j]))
    for cp in g_cps:
        cp.wait()

    def body(i, carry):
        ds = pl.ds(i * 16, 16)
        out_v[ds] = ((g_v[0][ds] + g_v[1][ds]) + (g_v[2][ds] + g_v[3][ds])) * 0.25
        return carry

    lax.fori_loop(0, CN // 16, body, 0)
    pltpu.sync_copy(out_v, out_hbm.at[pl.ds(base, CN)])


def kernel(array, length_of_link, node_at_link_head, node_at_link_tail, links_at_node):
    linksT = links_at_node.T.reshape(-1)
    grad = _grad_kernel(node_at_link_head, node_at_link_tail, length_of_link, array)
    return _slope_kernel(grad, linksT)


# speedup vs baseline: 116.0515x; 1.4660x over previous
import functools

import jax
import jax.numpy as jnp
from jax import lax
from jax.experimental import pallas as pl
from jax.experimental.pallas import tpu as pltpu
from jax.experimental.pallas import tpu_sc as plsc

N = 100000
L = 200000
K = 4

NC = 2
NS = 16
NW = NC * NS

CA = 6256
CL = 12512
CN = 3136

_mesh = plsc.VectorSubcoreMesh(core_axis_name="c", subcore_axis_name="s")
_params = pltpu.CompilerParams(needs_layout_passes=False)


@functools.partial(
    pl.kernel,
    out_type=jax.ShapeDtypeStruct((N,), jnp.float32),
    mesh=_mesh,
    compiler_params=_params,
    scratch_types=[
        pltpu.VMEM_SHARED((N,), jnp.float32),
        pltpu.VMEM_SHARED((L,), jnp.float32),
        pltpu.VMEM((CL,), jnp.int32),
        pltpu.VMEM((CL,), jnp.int32),
        pltpu.VMEM((CL,), jnp.float32),
        pltpu.VMEM((CL,), jnp.float32),
        pltpu.VMEM((CL,), jnp.float32),
        pltpu.VMEM((CL,), jnp.float32),
        [pltpu.VMEM((CN,), jnp.int32) for _ in range(K)],
        [pltpu.VMEM((CN,), jnp.float32) for _ in range(K)],
        pltpu.VMEM((CN,), jnp.float32),
        pltpu.SemaphoreType.DMA,
        pltpu.SemaphoreType.DMA,
        pltpu.SemaphoreType.DMA,
        pltpu.SemaphoreType.DMA,
    ],
)
def _slope_fused(head_hbm, tail_hbm, len_hbm, array_hbm, linksT_hbm, out_hbm,
                 arr_s, grad_s,
                 head_v, tail_v, len_v, hval_v, tval_v, grad_v,
                 links_v, g_v, out_v,
                 sem0, sem1, sem2, sem3):
    cid = lax.axis_index("c")
    sid = lax.axis_index("s")
    wid = sid * NC + cid

    abase = jnp.minimum(sid * CA, N - CA)
    pltpu.sync_copy(array_hbm.at[pl.ds(abase, CA)], grad_v.at[pl.ds(0, CA)])
    cp_a = pltpu.async_copy(grad_v.at[pl.ds(0, CA)],
                            arr_s.at[pl.ds(abase, CA)], sem3)
    lbase = jnp.minimum(sid * CL, L - CL)
    cp_h = pltpu.async_copy(head_hbm.at[pl.ds(lbase, CL)], head_v, sem0)
    cp_t = pltpu.async_copy(tail_hbm.at[pl.ds(lbase, CL)], tail_v, sem1)
    cp_l = pltpu.async_copy(len_hbm.at[pl.ds(lbase, CL)], len_v, sem2)
    cp_a.wait()
    cp_h.wait()
    cp_t.wait()
    plsc.subcore_barrier()
    g_h = pltpu.async_copy(arr_s.at[head_v], hval_v, sem0)
    g_t = pltpu.async_copy(arr_s.at[tail_v], tval_v, sem1)
    cp_l.wait()
    g_h.wait()
    g_t.wait()

    def body_a(i, carry):
        ds = pl.ds(i * 16, 16)
        grad_v[ds] = (hval_v[ds] - tval_v[ds]) / len_v[ds]
        return carry

    lax.fori_loop(0, CL // 16, body_a, 0)
    pltpu.sync_copy(grad_v, grad_s.at[pl.ds(lbase, CL)])
    plsc.subcore_barrier()

    nbase = jnp.minimum(wid * CN, N - CN)
    idx_cps = [
        pltpu.async_copy(linksT_hbm.at[pl.ds(j * N + nbase, CN)],
                         links_v[j], [sem0, sem1, sem2, sem3][j])
        for j in range(K)
    ]
    g_cps = []
    for j in range(K):
        idx_cps[j].wait()
        g_cps.append(pltpu.async_copy(grad_s.at[links_v[j]], g_v[j],
                                      [sem0, sem1, sem2, sem3][j]))
    for cp in g_cps:
        cp.wait()

    def body_b(i, carry):
        ds = pl.ds(i * 16, 16)
        out_v[ds] = ((g_v[0][ds] + g_v[1][ds]) + (g_v[2][ds] + g_v[3][ds])) * 0.25
        return carry

    lax.fori_loop(0, CN // 16, body_b, 0)
    pltpu.sync_copy(out_v, out_hbm.at[pl.ds(nbase, CN)])


def kernel(array, length_of_link, node_at_link_head, node_at_link_tail, links_at_node):
    linksT = links_at_node.T.reshape(-1)
    return _slope_fused(node_at_link_head, node_at_link_tail, length_of_link,
                        array, linksT)
